# Initial kernel scaffold; baseline (speedup 1.0000x reference)
import jax
import jax.numpy as jnp
from jax.experimental import pallas as pl


def kernel(x0, x1, adj00, adj01, adj10, adj11, W0, W1, a1_0, a2_0, a1_1, a2_1, Wl0, bl0, al0, Wl1, bl1, al1):
    raise NotImplementedError("write your pallas kernel here")


# speedup vs baseline: 1.7694x; 1.7694x over previous
import jax
import jax.numpy as jnp
from jax.experimental import pallas as pl

N = 5000
D = 128
H = 128
HS = 64
GAMMA = 0.1
BI = 200


def _lrelu(x):
    return jnp.where(x > 0, x, 0.2 * x)


def _proj_body(x0_ref, x1_ref, w0_ref, w1_ref, h0_ref, h1_ref):
    h0_ref[...] = jnp.dot(x0_ref[...], w0_ref[...],
                          preferred_element_type=jnp.float32)
    h1_ref[...] = jnp.dot(x1_ref[...], w1_ref[...],
                          preferred_element_type=jnp.float32)


def _hgat_body(hq_ref, h0_ref, h1_ref, adj0_ref, adj1_ref,
               a1s_ref, a2s_ref, wl_ref, bl_ref, al_ref, out_ref):
    hq = hq_ref[...]
    a1s = a1s_ref[...]
    a2s = a2s_ref[...]

    xts = []
    for t2 in range(2):
        g = h0_ref[...] if t2 == 0 else h1_ref[...]
        adj = adj0_ref[...] if t2 == 0 else adj1_ref[...]
        a1 = a1s[t2:t2 + 1, :]
        a2 = a2s[t2:t2 + 1, :]

        u = jnp.sum(hq * a1, axis=1, keepdims=True)
        v = jnp.sum(g * a2, axis=1)
        m = _lrelu(u + jnp.max(v))
        e = _lrelu(u + v[None, :])
        w = adj * jnp.exp(e - m)
        z = jnp.sum(w, axis=1, keepdims=True)

        scale = GAMMA / jnp.where(z > 0, z, 1.0)
        unif = jnp.where(z > 0, 0.0, GAMMA / N)
        att = w * scale + (1.0 - GAMMA) * adj + unif
        xts.append(jnp.dot(att, g, preferred_element_type=jnp.float32))

    wl = wl_ref[...]
    bl = bl_ref[...]
    al = al_ref[...]
    scores = []
    for t2 in range(2):
        s = jnp.tanh(jnp.dot(xts[t2], wl,
                             preferred_element_type=jnp.float32) + bl)
        scores.append(jnp.sum(s * al, axis=1, keepdims=True))
    smax = jnp.maximum(scores[0], scores[1])
    e0 = jnp.exp(scores[0] - smax)
    e1 = jnp.exp(scores[1] - smax)
    pooled = (e0 * xts[0] + e1 * xts[1]) / (e0 + e1)
    out_ref[...] = jnp.maximum(pooled, 0.0)


def kernel(x0, x1, adj00, adj01, adj10, adj11, W0, W1,
           a1_0, a2_0, a1_1, a2_1, Wl0, bl0, al0, Wl1, bl1, al1):
    nb = N // BI

    h0, h1 = pl.pallas_call(
        _proj_body,
        grid=(nb,),
        in_specs=[
            pl.BlockSpec((BI, D), lambda i: (i, 0)),
            pl.BlockSpec((BI, D), lambda i: (i, 0)),
            pl.BlockSpec((D, H), lambda i: (0, 0)),
            pl.BlockSpec((D, H), lambda i: (0, 0)),
        ],
        out_specs=[
            pl.BlockSpec((BI, H), lambda i: (i, 0)),
            pl.BlockSpec((BI, H), lambda i: (i, 0)),
        ],
        out_shape=[
            jax.ShapeDtypeStruct((N, H), jnp.float32),
            jax.ShapeDtypeStruct((N, H), jnp.float32),
        ],
    )(x0, x1, W0, W1)

    a1s = jnp.concatenate([a1_0.reshape(1, H), a1_1.reshape(1, H)], axis=0)
    a2s = jnp.concatenate([a2_0.reshape(1, H), a2_1.reshape(1, H)], axis=0)

    def run_t1(hq, adjA, adjB, wl, bl, al):
        return pl.pallas_call(
            _hgat_body,
            grid=(nb,),
            in_specs=[
                pl.BlockSpec((BI, H), lambda i: (i, 0)),
                pl.BlockSpec((N, H), lambda i: (0, 0)),
                pl.BlockSpec((N, H), lambda i: (0, 0)),
                pl.BlockSpec((BI, N), lambda i: (i, 0)),
                pl.BlockSpec((BI, N), lambda i: (i, 0)),
                pl.BlockSpec((2, H), lambda i: (0, 0)),
                pl.BlockSpec((2, H), lambda i: (0, 0)),
                pl.BlockSpec((H, HS), lambda i: (0, 0)),
                pl.BlockSpec((1, HS), lambda i: (0, 0)),
                pl.BlockSpec((1, HS), lambda i: (0, 0)),
            ],
            out_specs=pl.BlockSpec((BI, H), lambda i: (i, 0)),
            out_shape=jax.ShapeDtypeStruct((N, H), jnp.float32),
        )(hq, h0, h1, adjA, adjB, a1s, a2s, wl, bl, al)

    out0 = run_t1(h0, adj00, adj01, Wl0, bl0.reshape(1, HS), al0.reshape(1, HS))
    out1 = run_t1(h1, adj10, adj11, Wl1, bl1.reshape(1, HS), al1.reshape(1, HS))
    return (out0, out1)


# speedup vs baseline: 1.9417x; 1.0974x over previous
import jax
import jax.numpy as jnp
from jax.experimental import pallas as pl

N = 5000
D = 128
H = 128
HS = 64
GAMMA = 0.1
BI = 200


def _lrelu(x):
    return jnp.where(x > 0, x, 0.2 * x)


def _proj_body(x0_ref, x1_ref, w0_ref, w1_ref, h0_ref, h1_ref):
    h0_ref[...] = jnp.dot(x0_ref[...], w0_ref[...],
                          preferred_element_type=jnp.float32)
    h1_ref[...] = jnp.dot(x1_ref[...], w1_ref[...],
                          preferred_element_type=jnp.float32)


def _hgat_body(hq_ref, h0_ref, h1_ref, adj0_ref, adj1_ref,
               a1s_ref, a2s_ref, wl_ref, bl_ref, al_ref, out_ref):
    hq = hq_ref[...]
    a1s = a1s_ref[...]
    a2s = a2s_ref[...]

    xts = []
    for t2 in range(2):
        g = h0_ref[...] if t2 == 0 else h1_ref[...]
        adj = adj0_ref[...] if t2 == 0 else adj1_ref[...]
        a1 = a1s[t2:t2 + 1, :]
        a2 = a2s[t2:t2 + 1, :]

        u = jnp.sum(hq * a1, axis=1, keepdims=True)
        v = jnp.sum(g * a2, axis=1)
        t = u + v[None, :]
        e = jnp.exp(jnp.maximum(t, 0.2 * t))
        a09 = (1.0 - GAMMA) * adj
        w = a09 * e
        z = jnp.sum(w, axis=1, keepdims=True)
        scale = GAMMA / jnp.where(z > 0, z, 1.0)
        unif = jnp.where(z > 0, 0.0, GAMMA / N)
        att = w * scale + a09 + unif
        xts.append(jnp.dot(att, g, preferred_element_type=jnp.float32))

    wl = wl_ref[...]
    bl = bl_ref[...]
    al = al_ref[...]
    scores = []
    for t2 in range(2):
        s = jnp.tanh(jnp.dot(xts[t2], wl,
                             preferred_element_type=jnp.float32) + bl)
        scores.append(jnp.sum(s * al, axis=1, keepdims=True))
    smax = jnp.maximum(scores[0], scores[1])
    e0 = jnp.exp(scores[0] - smax)
    e1 = jnp.exp(scores[1] - smax)
    pooled = (e0 * xts[0] + e1 * xts[1]) / (e0 + e1)
    out_ref[...] = jnp.maximum(pooled, 0.0)


def kernel(x0, x1, adj00, adj01, adj10, adj11, W0, W1,
           a1_0, a2_0, a1_1, a2_1, Wl0, bl0, al0, Wl1, bl1, al1):
    nb = N // BI

    h0, h1 = pl.pallas_call(
        _proj_body,
        grid=(nb,),
        in_specs=[
            pl.BlockSpec((BI, D), lambda i: (i, 0)),
            pl.BlockSpec((BI, D), lambda i: (i, 0)),
            pl.BlockSpec((D, H), lambda i: (0, 0)),
            pl.BlockSpec((D, H), lambda i: (0, 0)),
        ],
        out_specs=[
            pl.BlockSpec((BI, H), lambda i: (i, 0)),
            pl.BlockSpec((BI, H), lambda i: (i, 0)),
        ],
        out_shape=[
            jax.ShapeDtypeStruct((N, H), jnp.float32),
            jax.ShapeDtypeStruct((N, H), jnp.float32),
        ],
    )(x0, x1, W0, W1)

    a1s = jnp.concatenate([a1_0.reshape(1, H), a1_1.reshape(1, H)], axis=0)
    a2s = jnp.concatenate([a2_0.reshape(1, H), a2_1.reshape(1, H)], axis=0)

    def run_t1(hq, adjA, adjB, wl, bl, al):
        return pl.pallas_call(
            _hgat_body,
            grid=(nb,),
            in_specs=[
                pl.BlockSpec((BI, H), lambda i: (i, 0)),
                pl.BlockSpec((N, H), lambda i: (0, 0)),
                pl.BlockSpec((N, H), lambda i: (0, 0)),
                pl.BlockSpec((BI, N), lambda i: (i, 0)),
                pl.BlockSpec((BI, N), lambda i: (i, 0)),
                pl.BlockSpec((2, H), lambda i: (0, 0)),
                pl.BlockSpec((2, H), lambda i: (0, 0)),
                pl.BlockSpec((H, HS), lambda i: (0, 0)),
                pl.BlockSpec((1, HS), lambda i: (0, 0)),
                pl.BlockSpec((1, HS), lambda i: (0, 0)),
            ],
            out_specs=pl.BlockSpec((BI, H), lambda i: (i, 0)),
            out_shape=jax.ShapeDtypeStruct((N, H), jnp.float32),
        )(hq, h0, h1, adjA, adjB, a1s, a2s, wl, bl, al)

    out0 = run_t1(h0, adj00, adj01, Wl0, bl0.reshape(1, HS), al0.reshape(1, HS))
    out1 = run_t1(h1, adj10, adj11, Wl1, bl1.reshape(1, HS), al1.reshape(1, HS))
    return (out0, out1)
